# baseline (speedup 1.0000x reference)
import jax
import jax.numpy as jnp
from jax import lax
from jax.experimental import pallas as pl
from jax.experimental.pallas import tpu as pltpu

B, T, IDIM = 8, 2048, 80
HDIM, CDIM, TNUM = 160, 16, 10
NLOOP = HDIM // CDIM
K2_MEAN = sum((k + 1) ** 2 for k in range(NLOOP)) / NLOOP
K1_MEAN = sum((k + 1) for k in range(NLOOP)) / NLOOP


def _loss_kernel(ilens_ref, x_ref, w1_ref, b1_ref, w2_ref, b2_ref, out_ref):
    g = pl.program_id(0)
    x = x_ref[0]

    h = jnp.tanh(
        lax.dot_general(w1_ref[...], x, (((1,), (0,)), ((), ())),
                        preferred_element_type=jnp.float32)
        + b1_ref[...]
    )
    e = (
        lax.dot_general(w2_ref[...], h, (((1,), (0,)), ((), ())),
                        preferred_element_type=jnp.float32)
        + b2_ref[...]
    )

    def s(a, i):
        return jnp.concatenate([a[:, i:], a[:, :i]], axis=1)

    u = s(x, 1) + s(x, 2)
    w = u + s(u, 2)
    w = w + s(w, 4)
    msum = w + s(u, 8)

    t_idx = lax.broadcasted_iota(jnp.int32, (IDIM, T), 1)
    vmask = (t_idx < (ilens_ref[g] - TNUM)).astype(jnp.float32)

    q = e * vmask
    pm = msum * vmask
    a_part = jnp.sum(q * e)
    b_part = jnp.sum(q * msum)
    c_part = jnp.sum(pm * msum)
    part = (K2_MEAN * a_part
            - (2.0 * K1_MEAN / TNUM) * b_part
            + (1.0 / (TNUM * TNUM)) * c_part)

    @pl.when(g == 0)
    def _():
        out_ref[0, 0] = 0.0

    out_ref[0, 0] += part


@jax.jit
def _run(xs_t, ilens, W1, b1, W2, b2):
    grid_spec = pltpu.PrefetchScalarGridSpec(
        num_scalar_prefetch=1,
        grid=(B,),
        in_specs=[
            pl.BlockSpec((1, IDIM, T), lambda g, ilens: (g, 0, 0)),
            pl.BlockSpec((HDIM, IDIM), lambda g, ilens: (0, 0)),
            pl.BlockSpec((HDIM, 1), lambda g, ilens: (0, 0)),
            pl.BlockSpec((IDIM, HDIM), lambda g, ilens: (0, 0)),
            pl.BlockSpec((IDIM, 1), lambda g, ilens: (0, 0)),
        ],
        out_specs=pl.BlockSpec(memory_space=pltpu.SMEM),
    )
    out = pl.pallas_call(
        _loss_kernel,
        grid_spec=grid_spec,
        out_shape=jax.ShapeDtypeStruct((1, 1), jnp.float32),
    )(ilens.astype(jnp.int32), xs_t,
      W1.T, b1.reshape(HDIM, 1), W2.T, b2.reshape(IDIM, 1))
    return out[0, 0]


def kernel(xs_pad, ilens, ys_pad, W1, b1, W2, b2):
    del ys_pad
    xs_t = jnp.transpose(xs_pad, (0, 2, 1))
    return _run(xs_t, ilens, W1, b1, W2, b2)


# speedup vs baseline: 1.1540x; 1.1540x over previous
import jax
import jax.numpy as jnp
from jax import lax
from jax.experimental import pallas as pl
from jax.experimental.pallas import tpu as pltpu

B, T, IDIM = 8, 2048, 80
HDIM, CDIM, TNUM = 160, 16, 10
NLOOP = HDIM // CDIM
K2_MEAN = sum((k + 1) ** 2 for k in range(NLOOP)) / NLOOP
K1_MEAN = sum((k + 1) for k in range(NLOOP)) / NLOOP


def _loss_kernel(ilens_ref, x_ref, w1_ref, b1_ref, w2_ref, b2_ref, out_ref):
    g = pl.program_id(0)
    x = x_ref[0]

    h = jnp.tanh(
        lax.dot_general(w1_ref[...], x, (((0,), (0,)), ((), ())),
                        preferred_element_type=jnp.float32)
        + b1_ref[...]
    )
    e = (
        lax.dot_general(w2_ref[...], h, (((0,), (0,)), ((), ())),
                        preferred_element_type=jnp.float32)
        + b2_ref[...]
    )

    def s(a, i):
        return jnp.concatenate([a[:, i:], a[:, :i]], axis=1)

    u = s(x, 1) + s(x, 2)
    w = u + s(u, 2)
    w = w + s(w, 4)
    msum = w + s(u, 8)

    t_idx = lax.broadcasted_iota(jnp.int32, (IDIM, T), 1)
    vmask = (t_idx < (ilens_ref[g] - TNUM)).astype(jnp.float32)

    q = e * vmask
    pm = msum * vmask
    a_part = jnp.sum(q * e)
    b_part = jnp.sum(q * msum)
    c_part = jnp.sum(pm * msum)
    part = (K2_MEAN * a_part
            - (2.0 * K1_MEAN / TNUM) * b_part
            + (1.0 / (TNUM * TNUM)) * c_part)

    @pl.when(g == 0)
    def _():
        out_ref[0, 0] = 0.0

    out_ref[0, 0] += part


@jax.jit
def _run(xs_t, ilens, W1, b1, W2, b2):
    grid_spec = pltpu.PrefetchScalarGridSpec(
        num_scalar_prefetch=1,
        grid=(B,),
        in_specs=[
            pl.BlockSpec((1, IDIM, T), lambda g, ilens: (g, 0, 0)),
            pl.BlockSpec((IDIM, HDIM), lambda g, ilens: (0, 0)),
            pl.BlockSpec((HDIM, 1), lambda g, ilens: (0, 0)),
            pl.BlockSpec((HDIM, IDIM), lambda g, ilens: (0, 0)),
            pl.BlockSpec((IDIM, 1), lambda g, ilens: (0, 0)),
        ],
        out_specs=pl.BlockSpec(memory_space=pltpu.SMEM),
    )
    out = pl.pallas_call(
        _loss_kernel,
        grid_spec=grid_spec,
        out_shape=jax.ShapeDtypeStruct((1, 1), jnp.float32),
    )(ilens.astype(jnp.int32), xs_t,
      W1, b1.reshape(HDIM, 1), W2, b2.reshape(IDIM, 1))
    return out[0, 0]


def kernel(xs_pad, ilens, ys_pad, W1, b1, W2, b2):
    del ys_pad
    xs_t = jnp.transpose(xs_pad, (0, 2, 1))
    return _run(xs_t, ilens, W1, b1, W2, b2)
